# baseline (speedup 1.0000x reference)
import jax
import jax.numpy as jnp
from jax.experimental import pallas as pl
from jax.experimental.pallas import tpu as pltpu

TOTAL_TOKENS = 32768
D = 1024
NUM_DMAS = 8
ROWS_PER_DMA = TOTAL_TOKENS // NUM_DMAS


def _dma_copy_body(x_hbm, o_hbm, sems):
    for k in range(NUM_DMAS):
        pltpu.make_async_copy(
            x_hbm.at[pl.ds(k * ROWS_PER_DMA, ROWS_PER_DMA)],
            o_hbm.at[pl.ds(k * ROWS_PER_DMA, ROWS_PER_DMA)],
            sems.at[k],
        ).start()
    for k in range(NUM_DMAS):
        pltpu.make_async_copy(
            x_hbm.at[pl.ds(k * ROWS_PER_DMA, ROWS_PER_DMA)],
            o_hbm.at[pl.ds(k * ROWS_PER_DMA, ROWS_PER_DMA)],
            sems.at[k],
        ).wait()


def kernel(flat, row_splits):
    values = pl.pallas_call(
        _dma_copy_body,
        in_specs=[pl.BlockSpec(memory_space=pl.ANY)],
        out_specs=pl.BlockSpec(memory_space=pl.ANY),
        out_shape=jax.ShapeDtypeStruct((TOTAL_TOKENS, D), jnp.float32),
        scratch_shapes=[pltpu.SemaphoreType.DMA((NUM_DMAS,))],
    )(flat)
    return (values, row_splits)


# speedup vs baseline: 47.5727x; 47.5727x over previous
import jax
import jax.numpy as jnp
from jax.experimental import pallas as pl
from jax.experimental.pallas import tpu as pltpu

TOTAL_TOKENS = 32768
D = 1024
BLOCK_ROWS = 1024


def _copy_body(x_ref, o_ref):
    o_ref[...] = x_ref[...]


def kernel(flat, row_splits):
    values = pl.pallas_call(
        _copy_body,
        grid=(TOTAL_TOKENS // BLOCK_ROWS,),
        in_specs=[pl.BlockSpec((BLOCK_ROWS, D), lambda i: (i, 0))],
        out_specs=pl.BlockSpec((BLOCK_ROWS, D), lambda i: (i, 0)),
        out_shape=jax.ShapeDtypeStruct((TOTAL_TOKENS, D), jnp.float32),
    )(flat)
    return (values, row_splits)


# speedup vs baseline: 48.5723x; 1.0210x over previous
import jax
import jax.numpy as jnp
from jax.experimental import pallas as pl
from jax.experimental.pallas import tpu as pltpu

TOTAL_TOKENS = 32768
D = 1024
BLOCK_ROWS = 2048


def _copy_body(x_ref, o_ref):
    o_ref[...] = x_ref[...]


def kernel(flat, row_splits):
    values = pl.pallas_call(
        _copy_body,
        grid=(TOTAL_TOKENS // BLOCK_ROWS,),
        in_specs=[pl.BlockSpec((BLOCK_ROWS, D), lambda i: (i, 0))],
        out_specs=pl.BlockSpec((BLOCK_ROWS, D), lambda i: (i, 0)),
        out_shape=jax.ShapeDtypeStruct((TOTAL_TOKENS, D), jnp.float32),
    )(flat)
    return (values, row_splits)
